# baseline (speedup 1.0000x reference)
import functools

import jax
import jax.numpy as jnp
from jax import lax
from jax.experimental import pallas as pl
from jax.experimental.pallas import tpu as pltpu
from jax.experimental.pallas import tpu_sc as plsc

_L = 16
_NC, _NS = 2, 16
_B1, _B2, _B3 = 12, 12, 8
_NB1, _NB2, _NB3 = 1 << _B1, 1 << _B2, 1 << _B3


def _sortable_key(x):
    b = lax.bitcast_convert_type(x, jnp.int32)
    return b ^ (lax.shift_right_arithmetic(b, 31) & jnp.int32(0x7FFFFFFF))


def _scan_level(hc, hs, nbins, target_rank):
    iot = lax.iota(jnp.int32, _L)
    zi = jnp.zeros((_L,), jnp.int32)
    zf = jnp.zeros((_L,), jnp.float32)

    def body(i, st):
        found, b_sel, c_incl, s_incl, binc, bins_, rc, rs = st
        sl = pl.ds(i * _L, _L)
        c = hc[sl]
        s = hs[sl]
        hc[sl] = zi
        hs[sl] = zf
        blk_c = jnp.sum(c)
        blk_s = jnp.sum(s)
        rc_new = rc + blk_c

        def take(_):
            cc = jnp.cumsum(c)
            ss = jnp.cumsum(s)
            tot = rc + cc
            hit = tot >= target_rank
            lane = jnp.min(jnp.where(hit, iot, jnp.int32(64)))
            selm = iot == lane
            return (jnp.int32(1),
                    i * _L + lane,
                    jnp.sum(jnp.where(selm, tot, 0)),
                    rs + jnp.sum(jnp.where(selm, ss, 0.0)),
                    jnp.sum(jnp.where(selm, c, 0)),
                    jnp.sum(jnp.where(selm, s, 0.0)))

        def skip(_):
            return (found, b_sel, c_incl, s_incl, binc, bins_)

        pick = jnp.logical_and(found == 0, rc_new >= target_rank)
        found, b_sel, c_incl, s_incl, binc, bins_ = lax.cond(pick, take, skip, 0)
        return (found, b_sel, c_incl, s_incl, binc, bins_, rc_new, rs + blk_s)

    st0 = (jnp.int32(0), jnp.int32(0), jnp.int32(0), jnp.float32(0),
           jnp.int32(0), jnp.float32(0), jnp.int32(0), jnp.float32(0))
    st = lax.fori_loop(0, nbins // _L, body, st0)
    return st[1], st[2], st[3], st[4], st[5], st[6], st[7]


def _sc_body(x_hbm, out_hbm, row_a, row_b, h1c, h1s, h2c, h2s, h3c, h3s,
             res_v, sem_a, sem_b, *, n, k_above, dup_thresh, rows_per_w):
    wid = lax.axis_index("s") * _NC + lax.axis_index("c")
    iot = lax.iota(jnp.int32, _L)
    ones_i = jnp.full((_L,), 1, jnp.int32)
    zi = jnp.zeros((_L,), jnp.int32)
    zf = jnp.zeros((_L,), jnp.float32)
    res_s = jnp.zeros((_L,), jnp.float32)
    res_c = jnp.ones((_L,), jnp.float32)
    nchunks = n // _L
    row0 = wid * rows_per_w
    bufs = (row_a, row_b)
    sems = (sem_a, sem_b)

    @plsc.parallel_loop(0, _NB1 // _L, unroll=4)
    def z12(i):
        sl = pl.ds(i * _L, _L)
        h1c[sl] = zi
        h1s[sl] = zf
        h2c[sl] = zi
        h2s[sl] = zf

    @plsc.parallel_loop(0, _NB3 // _L, unroll=4)
    def z3(i):
        sl = pl.ds(i * _L, _L)
        h3c[sl] = zi
        h3s[sl] = zf

    copies = [pltpu.async_copy(x_hbm.at[row0], row_a, sem_a)]
    for j in range(rows_per_w):
        if j + 1 < rows_per_w:
            copies.append(pltpu.async_copy(
                x_hbm.at[row0 + j + 1], bufs[(j + 1) % 2], sems[(j + 1) % 2]))
        copies[j].wait()
        row_v = bufs[j % 2]

        @plsc.parallel_loop(0, nchunks, unroll=8)
        def p1(i):
            x = row_v[pl.ds(i * _L, _L)]
            key = _sortable_key(x)
            idx = lax.shift_right_arithmetic(key, 32 - _B1) + jnp.int32(_NB1 // 2)
            plsc.addupdate_scatter(h1c, [idx], ones_i)
            plsc.addupdate_scatter(h1s, [idx], x)

        a1 = n - k_above + 1
        b1, c1, s1, bc1, bs1, t1c, t1s = _scan_level(h1c, h1s, _NB1, a1)
        g_c = t1c - c1
        g_s = t1s - s1
        k2 = k_above - g_c
        top1 = b1 - jnp.int32(_NB1 // 2)

        @plsc.parallel_loop(0, nchunks, unroll=8)
        def p2(i):
            x = row_v[pl.ds(i * _L, _L)]
            key = _sortable_key(x)
            m = lax.shift_right_arithmetic(key, 32 - _B1) == top1
            idx = lax.shift_right_logical(key, 32 - _B1 - _B2) & jnp.int32(_NB2 - 1)
            plsc.addupdate_scatter(h2c, [idx], ones_i, mask=m)
            plsc.addupdate_scatter(h2s, [idx], x, mask=m)

        a2 = bc1 - k2 + 1
        b2, c2, s2, bc2, bs2, t2c, t2s = _scan_level(h2c, h2s, _NB2, a2)
        g_c = g_c + (t2c - c2)
        g_s = g_s + (t2s - s2)
        k3 = k2 - (t2c - c2)
        pref2 = (top1 << _B2) | b2

        @plsc.parallel_loop(0, nchunks, unroll=8)
        def p3(i):
            x = row_v[pl.ds(i * _L, _L)]
            key = _sortable_key(x)
            m = lax.shift_right_arithmetic(key, _B3) == pref2
            idx = key & jnp.int32(_NB3 - 1)
            plsc.addupdate_scatter(h3c, [idx], ones_i, mask=m)
            plsc.addupdate_scatter(h3s, [idx], x, mask=m)

        a3 = bc2 - k3 + 1
        b3, c3, s3, bc3, bs3, t3c, t3s = _scan_level(h3c, h3s, _NB3, a3)
        c_gt = g_c + (t3c - c3)
        s_gt = g_s + (t3s - s3)
        c_ge = c_gt + bc3
        s_ge = s_gt + bs3

        dup = c_ge >= dup_thresh
        cnt = jnp.where(dup, c_gt, c_ge).astype(jnp.float32)
        ssum = jnp.where(dup, s_gt, s_ge)
        res_s = jnp.where(iot == j, ssum, res_s)
        res_c = jnp.where(iot == j, cnt, res_c)

    res_v[...] = res_s / res_c
    pltpu.sync_copy(res_v, out_hbm.at[wid])


@jax.jit
def kernel(patch_logits):
    b, n = patch_logits.shape
    q = (100 - 10) / 100.0
    i0 = int(q * (n - 1))
    k_above = n - i0 - 1
    nw = _NC * _NS
    rows_per_w = b // nw

    body = functools.partial(
        _sc_body, n=n, k_above=k_above, dup_thresh=n - i0,
        rows_per_w=rows_per_w)
    out = pl.kernel(
        body,
        out_type=jax.ShapeDtypeStruct((nw, _L), jnp.float32),
        mesh=plsc.VectorSubcoreMesh(core_axis_name="c", subcore_axis_name="s",
                                    num_cores=_NC, num_subcores=_NS),
        compiler_params=pltpu.CompilerParams(needs_layout_passes=False),
        scratch_types=[
            pltpu.VMEM((n,), jnp.float32),
            pltpu.VMEM((n,), jnp.float32),
            pltpu.VMEM((_NB1,), jnp.int32),
            pltpu.VMEM((_NB1,), jnp.float32),
            pltpu.VMEM((_NB2,), jnp.int32),
            pltpu.VMEM((_NB2,), jnp.float32),
            pltpu.VMEM((_NB3,), jnp.int32),
            pltpu.VMEM((_NB3,), jnp.float32),
            pltpu.VMEM((_L,), jnp.float32),
            pltpu.SemaphoreType.DMA,
            pltpu.SemaphoreType.DMA,
        ],
    )(patch_logits)
    return out[:, :rows_per_w].reshape(b, 1)


# speedup vs baseline: 1.4407x; 1.4407x over previous
import functools

import jax
import jax.numpy as jnp
from jax import lax
from jax.experimental import pallas as pl
from jax.experimental.pallas import tpu as pltpu
from jax.experimental.pallas import tpu_sc as plsc

_L = 16
_NC, _NS = 2, 16
_B1, _B2, _B3 = 12, 12, 8
_NB1, _NB2, _NB3 = 1 << _B1, 1 << _B2, 1 << _B3


def _sortable_key(x):
    b = lax.bitcast_convert_type(x, jnp.int32)
    return b ^ (lax.shift_right_arithmetic(b, 31) & jnp.int32(0x7FFFFFFF))


def _scan_level(hc, hs, nbins, target_rank):
    iot = lax.iota(jnp.int32, _L)
    zi = jnp.zeros((_L,), jnp.int32)
    zf = jnp.zeros((_L,), jnp.float32)

    def body(i, st):
        found, b_sel, c_incl, s_incl, binc, bins_, rc, rs = st
        sl = pl.ds(i * _L, _L)
        c = hc[sl]
        s = hs[sl]
        hc[sl] = zi
        hs[sl] = zf
        blk_c = jnp.sum(c)
        blk_s = jnp.sum(s)
        rc_new = rc + blk_c

        def take(_):
            cc = jnp.cumsum(c)
            ss = jnp.cumsum(s)
            tot = rc + cc
            hit = tot >= target_rank
            lane = jnp.min(jnp.where(hit, iot, jnp.int32(64)))
            selm = iot == lane
            return (jnp.int32(1),
                    i * _L + lane,
                    jnp.sum(jnp.where(selm, tot, 0)),
                    rs + jnp.sum(jnp.where(selm, ss, 0.0)),
                    jnp.sum(jnp.where(selm, c, 0)),
                    jnp.sum(jnp.where(selm, s, 0.0)))

        def skip(_):
            return (found, b_sel, c_incl, s_incl, binc, bins_)

        pick = jnp.logical_and(found == 0, rc_new >= target_rank)
        found, b_sel, c_incl, s_incl, binc, bins_ = lax.cond(pick, take, skip, 0)
        return (found, b_sel, c_incl, s_incl, binc, bins_, rc_new, rs + blk_s)

    st0 = (jnp.int32(0), jnp.int32(0), jnp.int32(0), jnp.float32(0),
           jnp.int32(0), jnp.float32(0), jnp.int32(0), jnp.float32(0))
    st = lax.fori_loop(0, nbins // _L, body, st0)
    return st[1], st[2], st[3], st[4], st[5], st[6], st[7]


def _sc_body(x_hbm, out_hbm, row_a, row_b, h1c, h1s, h2c, h2s, h3c, h3s,
             res_v, sem_a, sem_b, *, n, k_above, dup_thresh, rows_per_w):
    wid = lax.axis_index("s") * _NC + lax.axis_index("c")
    iot = lax.iota(jnp.int32, _L)
    ones_i = jnp.full((_L,), 1, jnp.int32)
    zi = jnp.zeros((_L,), jnp.int32)
    zf = jnp.zeros((_L,), jnp.float32)
    res_s = jnp.zeros((_L,), jnp.float32)
    res_c = jnp.ones((_L,), jnp.float32)
    nchunks = n // _L
    row0 = wid * rows_per_w
    bufs = (row_a, row_b)
    sems = (sem_a, sem_b)

    @plsc.parallel_loop(0, _NB1 // _L, unroll=4)
    def z12(i):
        sl = pl.ds(i * _L, _L)
        h1c[sl] = zi
        h1s[sl] = zf
        h2c[sl] = zi
        h2s[sl] = zf

    @plsc.parallel_loop(0, _NB3 // _L, unroll=4)
    def z3(i):
        sl = pl.ds(i * _L, _L)
        h3c[sl] = zi
        h3s[sl] = zf

    copies = [pltpu.async_copy(x_hbm.at[row0], row_a, sem_a)]
    for j in range(rows_per_w):
        if j + 1 < rows_per_w:
            copies.append(pltpu.async_copy(
                x_hbm.at[row0 + j + 1], bufs[(j + 1) % 2], sems[(j + 1) % 2]))
        copies[j].wait()
        row_v = bufs[j % 2]

        @plsc.parallel_loop(0, nchunks, unroll=8)
        def p1(i):
            x = row_v[pl.ds(i * _L, _L)]
            key = _sortable_key(x)
            idx = lax.shift_right_arithmetic(key, 32 - _B1) + jnp.int32(_NB1 // 2)
            plsc.addupdate_scatter(h1c, [idx], ones_i)
            plsc.addupdate_scatter(h1s, [idx], x)

        a1 = n - k_above + 1
        b1, c1, s1, bc1, bs1, t1c, t1s = _scan_level(h1c, h1s, _NB1, a1)
        g_c = t1c - c1
        g_s = t1s - s1
        k2 = k_above - g_c
        top1 = b1 - jnp.int32(_NB1 // 2)

        @plsc.parallel_loop(0, nchunks, unroll=8)
        def p2(i):
            x = row_v[pl.ds(i * _L, _L)]
            key = _sortable_key(x)
            m = lax.shift_right_arithmetic(key, 32 - _B1) == top1
            idx = lax.shift_right_logical(key, 32 - _B1 - _B2) & jnp.int32(_NB2 - 1)
            plsc.addupdate_scatter(h2c, [idx], ones_i, mask=m)
            plsc.addupdate_scatter(h2s, [idx], x, mask=m)

        a2 = bc1 - k2 + 1
        b2, c2, s2, bc2, bs2, t2c, t2s = _scan_level(h2c, h2s, _NB2, a2)
        g_c = g_c + (t2c - c2)
        g_s = g_s + (t2s - s2)
        k3 = k2 - (t2c - c2)
        pref2 = (top1 << _B2) | b2

        @plsc.parallel_loop(0, nchunks, unroll=8)
        def p3(i):
            x = row_v[pl.ds(i * _L, _L)]
            key = _sortable_key(x)
            m = lax.shift_right_arithmetic(key, _B3) == pref2
            idx = key & jnp.int32(_NB3 - 1)
            plsc.addupdate_scatter(h3c, [idx], ones_i, mask=m)
            plsc.addupdate_scatter(h3s, [idx], x, mask=m)

        a3 = bc2 - k3 + 1
        b3, c3, s3, bc3, bs3, t3c, t3s = _scan_level(h3c, h3s, _NB3, a3)
        c_gt = g_c + (t3c - c3)
        s_gt = g_s + (t3s - s3)
        c_ge = c_gt + bc3
        s_ge = s_gt + bs3

        dup = c_ge >= dup_thresh
        cnt = jnp.where(dup, c_gt, c_ge).astype(jnp.float32)
        ssum = jnp.where(dup, s_gt, s_ge)
        res_s = jnp.where(iot == j, ssum, res_s)
        res_c = jnp.where(iot == j, cnt, res_c)

    res_v[...] = res_s / res_c
    pltpu.sync_copy(res_v, out_hbm.at[wid])


def _tc_body(x_ref, out_ref, *, k_above, n_above_lo):
    x = x_ref[...]
    bits = jax.lax.bitcast_convert_type(x, jnp.int32)
    key = bits ^ (lax.shift_right_arithmetic(bits, 31) & jnp.int32(0x7FFFFFFF))
    t0 = jnp.full((x.shape[0], 1), jnp.int32(-(2 ** 31)), dtype=jnp.int32)

    def step(i, t):
        cand = t + jnp.left_shift(jnp.int32(1), jnp.int32(31) - i)
        cnt = jnp.sum((key >= cand).astype(jnp.int32), axis=1, keepdims=True)
        return jnp.where(cnt >= k_above, cand, t)

    t = lax.fori_loop(0, 32, step, t0)
    ge = key >= t
    gt = key > t
    c_ge = jnp.sum(ge.astype(jnp.int32), axis=1, keepdims=True)
    c_gt = jnp.sum(gt.astype(jnp.int32), axis=1, keepdims=True)
    s_ge = jnp.sum(jnp.where(ge, x, 0.0), axis=1, keepdims=True)
    s_gt = jnp.sum(jnp.where(gt, x, 0.0), axis=1, keepdims=True)
    dup = c_ge >= n_above_lo
    cnt = jnp.where(dup, c_gt, c_ge).astype(jnp.float32)
    s = jnp.where(dup, s_gt, s_ge)
    out_ref[...] = s / cnt


_SC_ROWS = 64


@jax.jit
def kernel(patch_logits):
    b, n = patch_logits.shape
    q = (100 - 10) / 100.0
    i0 = int(q * (n - 1))
    k_above = n - i0 - 1
    nw = _NC * _NS
    rows_per_w = _SC_ROWS // nw

    body = functools.partial(
        _sc_body, n=n, k_above=k_above, dup_thresh=n - i0,
        rows_per_w=rows_per_w)
    out_sc = pl.kernel(
        body,
        out_type=jax.ShapeDtypeStruct((nw, _L), jnp.float32),
        mesh=plsc.VectorSubcoreMesh(core_axis_name="c", subcore_axis_name="s",
                                    num_cores=_NC, num_subcores=_NS),
        compiler_params=pltpu.CompilerParams(needs_layout_passes=False),
        scratch_types=[
            pltpu.VMEM((n,), jnp.float32),
            pltpu.VMEM((n,), jnp.float32),
            pltpu.VMEM((_NB1,), jnp.int32),
            pltpu.VMEM((_NB1,), jnp.float32),
            pltpu.VMEM((_NB2,), jnp.int32),
            pltpu.VMEM((_NB2,), jnp.float32),
            pltpu.VMEM((_NB3,), jnp.int32),
            pltpu.VMEM((_NB3,), jnp.float32),
            pltpu.VMEM((_L,), jnp.float32),
            pltpu.SemaphoreType.DMA,
            pltpu.SemaphoreType.DMA,
        ],
    )(patch_logits[:_SC_ROWS])
    out_sc = out_sc[:, :rows_per_w].reshape(_SC_ROWS, 1)

    tc_rows = b - _SC_ROWS
    rows_per_block = 16
    out_tc = pl.pallas_call(
        functools.partial(_tc_body, k_above=k_above, n_above_lo=n - i0),
        grid=(tc_rows // rows_per_block,),
        in_specs=[pl.BlockSpec((rows_per_block, n), lambda i: (i, 0))],
        out_specs=pl.BlockSpec((rows_per_block, 1), lambda i: (i, 0)),
        out_shape=jax.ShapeDtypeStruct((tc_rows, 1), jnp.float32),
    )(patch_logits[_SC_ROWS:])

    return jnp.concatenate([out_sc, out_tc], axis=0)


# speedup vs baseline: 1.6082x; 1.1163x over previous
import functools

import jax
import jax.numpy as jnp
from jax import lax
from jax.experimental import pallas as pl
from jax.experimental.pallas import tpu as pltpu
from jax.experimental.pallas import tpu_sc as plsc

_L = 16
_NC, _NS = 2, 16
_B1, _B2, _B3 = 12, 12, 8
_NB1, _NB2, _NB3 = 1 << _B1, 1 << _B2, 1 << _B3


def _sortable_key(x):
    b = lax.bitcast_convert_type(x, jnp.int32)
    return b ^ (lax.shift_right_arithmetic(b, 31) & jnp.int32(0x7FFFFFFF))


def _scan_level(hc, hs, nbins, target_rank):
    iot = lax.iota(jnp.int32, _L)
    zi = jnp.zeros((_L,), jnp.int32)
    zf = jnp.zeros((_L,), jnp.float32)

    def body(i, st):
        found, b_sel, c_incl, s_incl, binc, bins_, rc, rs = st
        sl = pl.ds(i * _L, _L)
        c = hc[sl]
        s = hs[sl]
        hc[sl] = zi
        hs[sl] = zf
        blk_c = jnp.sum(c)
        blk_s = jnp.sum(s)
        rc_new = rc + blk_c

        def take(_):
            cc = jnp.cumsum(c)
            ss = jnp.cumsum(s)
            tot = rc + cc
            hit = tot >= target_rank
            lane = jnp.min(jnp.where(hit, iot, jnp.int32(64)))
            selm = iot == lane
            return (jnp.int32(1),
                    i * _L + lane,
                    jnp.sum(jnp.where(selm, tot, 0)),
                    rs + jnp.sum(jnp.where(selm, ss, 0.0)),
                    jnp.sum(jnp.where(selm, c, 0)),
                    jnp.sum(jnp.where(selm, s, 0.0)))

        def skip(_):
            return (found, b_sel, c_incl, s_incl, binc, bins_)

        pick = jnp.logical_and(found == 0, rc_new >= target_rank)
        found, b_sel, c_incl, s_incl, binc, bins_ = lax.cond(pick, take, skip, 0)
        return (found, b_sel, c_incl, s_incl, binc, bins_, rc_new, rs + blk_s)

    st0 = (jnp.int32(0), jnp.int32(0), jnp.int32(0), jnp.float32(0),
           jnp.int32(0), jnp.float32(0), jnp.int32(0), jnp.float32(0))
    st = lax.fori_loop(0, nbins // _L, body, st0)
    return st[1], st[2], st[3], st[4], st[5], st[6], st[7]


def _sc_body(x_hbm, out_hbm, row_a, row_b, h1c, h1s, h2c, h2s, h3c, h3s,
             res_v, sem_a, sem_b, *, n, k_above, dup_thresh, rows_per_w):
    wid = lax.axis_index("s") * _NC + lax.axis_index("c")
    iot = lax.iota(jnp.int32, _L)
    ones_i = jnp.full((_L,), 1, jnp.int32)
    zi = jnp.zeros((_L,), jnp.int32)
    zf = jnp.zeros((_L,), jnp.float32)
    res_s = jnp.zeros((_L,), jnp.float32)
    res_c = jnp.ones((_L,), jnp.float32)
    nchunks = n // _L
    row0 = wid * rows_per_w
    bufs = (row_a, row_b)
    sems = (sem_a, sem_b)

    @plsc.parallel_loop(0, _NB1 // _L, unroll=4)
    def z12(i):
        sl = pl.ds(i * _L, _L)
        h1c[sl] = zi
        h1s[sl] = zf
        h2c[sl] = zi
        h2s[sl] = zf

    @plsc.parallel_loop(0, _NB3 // _L, unroll=4)
    def z3(i):
        sl = pl.ds(i * _L, _L)
        h3c[sl] = zi
        h3s[sl] = zf

    copies = [pltpu.async_copy(x_hbm.at[row0], row_a, sem_a)]
    for j in range(rows_per_w):
        if j + 1 < rows_per_w:
            copies.append(pltpu.async_copy(
                x_hbm.at[row0 + j + 1], bufs[(j + 1) % 2], sems[(j + 1) % 2]))
        copies[j].wait()
        row_v = bufs[j % 2]

        @plsc.parallel_loop(0, nchunks, unroll=8)
        def p1(i):
            x = row_v[pl.ds(i * _L, _L)]
            key = _sortable_key(x)
            idx = lax.shift_right_arithmetic(key, 32 - _B1) + jnp.int32(_NB1 // 2)
            plsc.addupdate_scatter(h1c, [idx], ones_i)
            plsc.addupdate_scatter(h1s, [idx], x)

        a1 = n - k_above + 1
        b1, c1, s1, bc1, bs1, t1c, t1s = _scan_level(h1c, h1s, _NB1, a1)
        g_c = t1c - c1
        g_s = t1s - s1
        k2 = k_above - g_c
        top1 = b1 - jnp.int32(_NB1 // 2)

        @plsc.parallel_loop(0, nchunks, unroll=8)
        def p2(i):
            x = row_v[pl.ds(i * _L, _L)]
            key = _sortable_key(x)
            m = lax.shift_right_arithmetic(key, 32 - _B1) == top1
            idx = lax.shift_right_logical(key, 32 - _B1 - _B2) & jnp.int32(_NB2 - 1)
            plsc.addupdate_scatter(h2c, [idx], ones_i, mask=m)
            plsc.addupdate_scatter(h2s, [idx], x, mask=m)

        a2 = bc1 - k2 + 1
        b2, c2, s2, bc2, bs2, t2c, t2s = _scan_level(h2c, h2s, _NB2, a2)
        g_c = g_c + (t2c - c2)
        g_s = g_s + (t2s - s2)
        k3 = k2 - (t2c - c2)
        pref2 = (top1 << _B2) | b2

        @plsc.parallel_loop(0, nchunks, unroll=8)
        def p3(i):
            x = row_v[pl.ds(i * _L, _L)]
            key = _sortable_key(x)
            m = lax.shift_right_arithmetic(key, _B3) == pref2
            idx = key & jnp.int32(_NB3 - 1)
            plsc.addupdate_scatter(h3c, [idx], ones_i, mask=m)
            plsc.addupdate_scatter(h3s, [idx], x, mask=m)

        a3 = bc2 - k3 + 1
        b3, c3, s3, bc3, bs3, t3c, t3s = _scan_level(h3c, h3s, _NB3, a3)
        c_gt = g_c + (t3c - c3)
        s_gt = g_s + (t3s - s3)
        c_ge = c_gt + bc3
        s_ge = s_gt + bs3

        dup = c_ge >= dup_thresh
        cnt = jnp.where(dup, c_gt, c_ge).astype(jnp.float32)
        ssum = jnp.where(dup, s_gt, s_ge)
        res_s = jnp.where(iot == j, ssum, res_s)
        res_c = jnp.where(iot == j, cnt, res_c)

    res_v[...] = res_s / res_c
    pltpu.sync_copy(res_v, out_hbm.at[wid])


def _tc_body(x_ref, out_ref, *, k_above, n_above_lo):
    x = x_ref[...]
    bits = jax.lax.bitcast_convert_type(x, jnp.int32)
    key = bits ^ (lax.shift_right_arithmetic(bits, 31) & jnp.int32(0x7FFFFFFF))
    t0 = jnp.full((x.shape[0], 1), jnp.int32(-(2 ** 31)), dtype=jnp.int32)

    def step(i, t):
        cand = t + jnp.left_shift(jnp.int32(1), jnp.int32(31) - i)
        cnt = jnp.sum((key >= cand).astype(jnp.int32), axis=1, keepdims=True)
        return jnp.where(cnt >= k_above, cand, t)

    t = lax.fori_loop(0, 32, step, t0)
    ge = key >= t
    gt = key > t
    c_ge = jnp.sum(ge.astype(jnp.int32), axis=1, keepdims=True)
    c_gt = jnp.sum(gt.astype(jnp.int32), axis=1, keepdims=True)
    s_ge = jnp.sum(jnp.where(ge, x, 0.0), axis=1, keepdims=True)
    s_gt = jnp.sum(jnp.where(gt, x, 0.0), axis=1, keepdims=True)
    dup = c_ge >= n_above_lo
    cnt = jnp.where(dup, c_gt, c_ge).astype(jnp.float32)
    s = jnp.where(dup, s_gt, s_ge)
    out_ref[...] = s / cnt


_SC_ROWS = 64


@jax.jit
def kernel(patch_logits):
    b, n = patch_logits.shape
    q = (100 - 10) / 100.0
    i0 = int(q * (n - 1))
    k_above = n - i0 - 1
    nw = _NC * _NS
    rows_per_w = _SC_ROWS // nw

    body = functools.partial(
        _sc_body, n=n, k_above=k_above, dup_thresh=n - i0,
        rows_per_w=rows_per_w)
    out_sc = pl.kernel(
        body,
        out_type=jax.ShapeDtypeStruct((nw, _L), jnp.float32),
        mesh=plsc.VectorSubcoreMesh(core_axis_name="c", subcore_axis_name="s",
                                    num_cores=_NC, num_subcores=_NS),
        compiler_params=pltpu.CompilerParams(needs_layout_passes=False),
        scratch_types=[
            pltpu.VMEM((n,), jnp.float32),
            pltpu.VMEM((n,), jnp.float32),
            pltpu.VMEM((_NB1,), jnp.int32),
            pltpu.VMEM((_NB1,), jnp.float32),
            pltpu.VMEM((_NB2,), jnp.int32),
            pltpu.VMEM((_NB2,), jnp.float32),
            pltpu.VMEM((_NB3,), jnp.int32),
            pltpu.VMEM((_NB3,), jnp.float32),
            pltpu.VMEM((_L,), jnp.float32),
            pltpu.SemaphoreType.DMA,
            pltpu.SemaphoreType.DMA,
        ],
    )(patch_logits)
    out_sc = out_sc[:, :rows_per_w].reshape(_SC_ROWS, 1)

    tc_rows = b - _SC_ROWS
    rows_per_block = 16
    blk0 = _SC_ROWS // rows_per_block
    out_tc = pl.pallas_call(
        functools.partial(_tc_body, k_above=k_above, n_above_lo=n - i0),
        grid=(tc_rows // rows_per_block,),
        in_specs=[pl.BlockSpec((rows_per_block, n), lambda i: (i + blk0, 0))],
        out_specs=pl.BlockSpec((rows_per_block, 1), lambda i: (i, 0)),
        out_shape=jax.ShapeDtypeStruct((tc_rows, 1), jnp.float32),
    )(patch_logits)

    return jnp.concatenate([out_sc, out_tc], axis=0)


# speedup vs baseline: 1.6300x; 1.0135x over previous
import functools

import jax
import jax.numpy as jnp
from jax import lax
from jax.experimental import pallas as pl
from jax.experimental.pallas import tpu as pltpu
from jax.experimental.pallas import tpu_sc as plsc

_L = 16
_NC, _NS = 2, 16
_B1, _B2, _B3 = 11, 11, 10
_NB1, _NB2, _NB3 = 1 << _B1, 1 << _B2, 1 << _B3


def _sortable_key(x):
    b = lax.bitcast_convert_type(x, jnp.int32)
    return b ^ (lax.shift_right_arithmetic(b, 31) & jnp.int32(0x7FFFFFFF))


def _scan_level(hc, hs, nbins, target_rank):
    iot = lax.iota(jnp.int32, _L)
    zi = jnp.zeros((_L,), jnp.int32)
    zf = jnp.zeros((_L,), jnp.float32)

    def body(i, st):
        found, b_sel, c_incl, s_incl, binc, bins_, rc, rs = st
        sl = pl.ds(i * _L, _L)
        c = hc[sl]
        s = hs[sl]
        hc[sl] = zi
        hs[sl] = zf
        blk_c = jnp.sum(c)
        blk_s = jnp.sum(s)
        rc_new = rc + blk_c

        def take(_):
            cc = jnp.cumsum(c)
            ss = jnp.cumsum(s)
            tot = rc + cc
            hit = tot >= target_rank
            lane = jnp.min(jnp.where(hit, iot, jnp.int32(64)))
            selm = iot == lane
            return (jnp.int32(1),
                    i * _L + lane,
                    jnp.sum(jnp.where(selm, tot, 0)),
                    rs + jnp.sum(jnp.where(selm, ss, 0.0)),
                    jnp.sum(jnp.where(selm, c, 0)),
                    jnp.sum(jnp.where(selm, s, 0.0)))

        def skip(_):
            return (found, b_sel, c_incl, s_incl, binc, bins_)

        pick = jnp.logical_and(found == 0, rc_new >= target_rank)
        found, b_sel, c_incl, s_incl, binc, bins_ = lax.cond(pick, take, skip, 0)
        return (found, b_sel, c_incl, s_incl, binc, bins_, rc_new, rs + blk_s)

    st0 = (jnp.int32(0), jnp.int32(0), jnp.int32(0), jnp.float32(0),
           jnp.int32(0), jnp.float32(0), jnp.int32(0), jnp.float32(0))
    st = lax.fori_loop(0, nbins // _L, body, st0)
    return st[1], st[2], st[3], st[4], st[5], st[6], st[7]


def _sc_body(x_hbm, out_hbm, row_a, row_b, h1c, h1s, h2c, h2s, h3c, h3s,
             res_v, sem_a, sem_b, *, n, k_above, dup_thresh, rows_per_w):
    wid = lax.axis_index("s") * _NC + lax.axis_index("c")
    iot = lax.iota(jnp.int32, _L)
    ones_i = jnp.full((_L,), 1, jnp.int32)
    zi = jnp.zeros((_L,), jnp.int32)
    zf = jnp.zeros((_L,), jnp.float32)
    res_s = jnp.zeros((_L,), jnp.float32)
    res_c = jnp.ones((_L,), jnp.float32)
    nchunks = n // _L
    row0 = wid * rows_per_w
    bufs = (row_a, row_b)
    sems = (sem_a, sem_b)

    @plsc.parallel_loop(0, _NB1 // _L, unroll=4)
    def z12(i):
        sl = pl.ds(i * _L, _L)
        h1c[sl] = zi
        h1s[sl] = zf
        h2c[sl] = zi
        h2s[sl] = zf

    @plsc.parallel_loop(0, _NB3 // _L, unroll=4)
    def z3(i):
        sl = pl.ds(i * _L, _L)
        h3c[sl] = zi
        h3s[sl] = zf

    copies = [pltpu.async_copy(x_hbm.at[row0], row_a, sem_a)]
    for j in range(rows_per_w):
        if j + 1 < rows_per_w:
            copies.append(pltpu.async_copy(
                x_hbm.at[row0 + j + 1], bufs[(j + 1) % 2], sems[(j + 1) % 2]))
        copies[j].wait()
        row_v = bufs[j % 2]

        @plsc.parallel_loop(0, nchunks, unroll=8)
        def p1(i):
            x = row_v[pl.ds(i * _L, _L)]
            key = _sortable_key(x)
            idx = lax.shift_right_arithmetic(key, 32 - _B1) + jnp.int32(_NB1 // 2)
            plsc.addupdate_scatter(h1c, [idx], ones_i)
            plsc.addupdate_scatter(h1s, [idx], x)

        a1 = n - k_above + 1
        b1, c1, s1, bc1, bs1, t1c, t1s = _scan_level(h1c, h1s, _NB1, a1)
        g_c = t1c - c1
        g_s = t1s - s1
        k2 = k_above - g_c
        top1 = b1 - jnp.int32(_NB1 // 2)

        @plsc.parallel_loop(0, nchunks, unroll=8)
        def p2(i):
            x = row_v[pl.ds(i * _L, _L)]
            key = _sortable_key(x)
            m = lax.shift_right_arithmetic(key, 32 - _B1) == top1
            idx = lax.shift_right_logical(key, 32 - _B1 - _B2) & jnp.int32(_NB2 - 1)
            plsc.addupdate_scatter(h2c, [idx], ones_i, mask=m)
            plsc.addupdate_scatter(h2s, [idx], x, mask=m)

        a2 = bc1 - k2 + 1
        b2, c2, s2, bc2, bs2, t2c, t2s = _scan_level(h2c, h2s, _NB2, a2)
        g_c = g_c + (t2c - c2)
        g_s = g_s + (t2s - s2)
        k3 = k2 - (t2c - c2)
        pref2 = (top1 << _B2) | b2

        @plsc.parallel_loop(0, nchunks, unroll=8)
        def p3(i):
            x = row_v[pl.ds(i * _L, _L)]
            key = _sortable_key(x)
            m = lax.shift_right_arithmetic(key, _B3) == pref2
            idx = key & jnp.int32(_NB3 - 1)
            plsc.addupdate_scatter(h3c, [idx], ones_i, mask=m)
            plsc.addupdate_scatter(h3s, [idx], x, mask=m)

        a3 = bc2 - k3 + 1
        b3, c3, s3, bc3, bs3, t3c, t3s = _scan_level(h3c, h3s, _NB3, a3)
        c_gt = g_c + (t3c - c3)
        s_gt = g_s + (t3s - s3)
        c_ge = c_gt + bc3
        s_ge = s_gt + bs3

        dup = c_ge >= dup_thresh
        cnt = jnp.where(dup, c_gt, c_ge).astype(jnp.float32)
        ssum = jnp.where(dup, s_gt, s_ge)
        res_s = jnp.where(iot == j, ssum, res_s)
        res_c = jnp.where(iot == j, cnt, res_c)

    res_v[...] = res_s / res_c
    pltpu.sync_copy(res_v, out_hbm.at[wid])


def _tc_body(x_ref, out_ref, *, k_above, n_above_lo):
    x = x_ref[...]
    bits = jax.lax.bitcast_convert_type(x, jnp.int32)
    key = bits ^ (lax.shift_right_arithmetic(bits, 31) & jnp.int32(0x7FFFFFFF))
    t0 = jnp.full((x.shape[0], 1), jnp.int32(-(2 ** 31)), dtype=jnp.int32)

    def step(i, t):
        cand = t + jnp.left_shift(jnp.int32(1), jnp.int32(31) - i)
        cnt = jnp.sum((key >= cand).astype(jnp.int32), axis=1, keepdims=True)
        return jnp.where(cnt >= k_above, cand, t)

    t = lax.fori_loop(0, 32, step, t0)
    ge = key >= t
    gt = key > t
    c_ge = jnp.sum(ge.astype(jnp.int32), axis=1, keepdims=True)
    c_gt = jnp.sum(gt.astype(jnp.int32), axis=1, keepdims=True)
    s_ge = jnp.sum(jnp.where(ge, x, 0.0), axis=1, keepdims=True)
    s_gt = jnp.sum(jnp.where(gt, x, 0.0), axis=1, keepdims=True)
    dup = c_ge >= n_above_lo
    cnt = jnp.where(dup, c_gt, c_ge).astype(jnp.float32)
    s = jnp.where(dup, s_gt, s_ge)
    out_ref[...] = s / cnt


_SC_ROWS = 64


@jax.jit
def kernel(patch_logits):
    b, n = patch_logits.shape
    q = (100 - 10) / 100.0
    i0 = int(q * (n - 1))
    k_above = n - i0 - 1
    nw = _NC * _NS
    rows_per_w = _SC_ROWS // nw

    body = functools.partial(
        _sc_body, n=n, k_above=k_above, dup_thresh=n - i0,
        rows_per_w=rows_per_w)
    out_sc = pl.kernel(
        body,
        out_type=jax.ShapeDtypeStruct((nw, _L), jnp.float32),
        mesh=plsc.VectorSubcoreMesh(core_axis_name="c", subcore_axis_name="s",
                                    num_cores=_NC, num_subcores=_NS),
        compiler_params=pltpu.CompilerParams(needs_layout_passes=False),
        scratch_types=[
            pltpu.VMEM((n,), jnp.float32),
            pltpu.VMEM((n,), jnp.float32),
            pltpu.VMEM((_NB1,), jnp.int32),
            pltpu.VMEM((_NB1,), jnp.float32),
            pltpu.VMEM((_NB2,), jnp.int32),
            pltpu.VMEM((_NB2,), jnp.float32),
            pltpu.VMEM((_NB3,), jnp.int32),
            pltpu.VMEM((_NB3,), jnp.float32),
            pltpu.VMEM((_L,), jnp.float32),
            pltpu.SemaphoreType.DMA,
            pltpu.SemaphoreType.DMA,
        ],
    )(patch_logits)
    out_sc = out_sc[:, :rows_per_w].reshape(_SC_ROWS, 1)

    tc_rows = b - _SC_ROWS
    rows_per_block = 16
    blk0 = _SC_ROWS // rows_per_block
    out_tc = pl.pallas_call(
        functools.partial(_tc_body, k_above=k_above, n_above_lo=n - i0),
        grid=(tc_rows // rows_per_block,),
        in_specs=[pl.BlockSpec((rows_per_block, n), lambda i: (i + blk0, 0))],
        out_specs=pl.BlockSpec((rows_per_block, 1), lambda i: (i, 0)),
        out_shape=jax.ShapeDtypeStruct((tc_rows, 1), jnp.float32),
    )(patch_logits)

    return jnp.concatenate([out_sc, out_tc], axis=0)


# speedup vs baseline: 1.6316x; 1.0010x over previous
import functools

import jax
import jax.numpy as jnp
from jax import lax
from jax.experimental import pallas as pl
from jax.experimental.pallas import tpu as pltpu
from jax.experimental.pallas import tpu_sc as plsc

_L = 16
_NC, _NS = 2, 16
_B1, _B2, _B3 = 11, 11, 10
_NB1, _NB2, _NB3 = 1 << _B1, 1 << _B2, 1 << _B3


def _sortable_key(x):
    b = lax.bitcast_convert_type(x, jnp.int32)
    return b ^ (lax.shift_right_arithmetic(b, 31) & jnp.int32(0x7FFFFFFF))


def _scan_level(hc, hs, nbins, target_rank):
    iot = lax.iota(jnp.int32, _L)
    zi = jnp.zeros((_L,), jnp.int32)
    zf = jnp.zeros((_L,), jnp.float32)
    bpsb = nbins // _L // _L

    def sb_tot(sb, st):
        sbt_c, sbt_s = st
        acc_c = zi
        acc_s = zf
        for u in range(bpsb):
            sl = pl.ds((sb * bpsb + u) * _L, _L)
            acc_c = acc_c + hc[sl]
            acc_s = acc_s + hs[sl]
        sel = iot == sb
        return (jnp.where(sel, jnp.sum(acc_c), sbt_c),
                jnp.where(sel, jnp.sum(acc_s), sbt_s))

    sbt_c, sbt_s = lax.fori_loop(0, _L, sb_tot, (zi, zf))

    cum_c = jnp.cumsum(sbt_c)
    cum_s = jnp.cumsum(sbt_s)
    hitv = cum_c >= target_rank
    sbl = jnp.min(jnp.where(hitv, iot, jnp.int32(64)))
    selm = iot == sbl
    rc0 = jnp.sum(jnp.where(selm, cum_c - sbt_c, 0))
    rs0 = jnp.sum(jnp.where(selm, cum_s - sbt_s, 0.0))
    totc = jnp.sum(sbt_c)
    tots = jnp.sum(sbt_s)

    def body(u, st):
        found, b_sel, c_incl, s_incl, binc, bins_, rc, rs = st
        i = sbl * bpsb + u
        sl = pl.ds(i * _L, _L)
        c = hc[sl]
        s = hs[sl]
        blk_c = jnp.sum(c)
        blk_s = jnp.sum(s)
        rc_new = rc + blk_c

        def take(_):
            cc = jnp.cumsum(c)
            ss = jnp.cumsum(s)
            tot = rc + cc
            hit = tot >= target_rank
            lane = jnp.min(jnp.where(hit, iot, jnp.int32(64)))
            selb = iot == lane
            return (jnp.int32(1),
                    i * _L + lane,
                    jnp.sum(jnp.where(selb, tot, 0)),
                    rs + jnp.sum(jnp.where(selb, ss, 0.0)),
                    jnp.sum(jnp.where(selb, c, 0)),
                    jnp.sum(jnp.where(selb, s, 0.0)))

        def skip(_):
            return (found, b_sel, c_incl, s_incl, binc, bins_)

        pick = jnp.logical_and(found == 0, rc_new >= target_rank)
        found, b_sel, c_incl, s_incl, binc, bins_ = lax.cond(pick, take, skip, 0)
        return (found, b_sel, c_incl, s_incl, binc, bins_, rc_new, rs + blk_s)

    st0 = (jnp.int32(0), jnp.int32(0), jnp.int32(0), jnp.float32(0),
           jnp.int32(0), jnp.float32(0), rc0, rs0)
    st = lax.fori_loop(0, bpsb, body, st0)

    @plsc.parallel_loop(0, nbins // _L, unroll=4)
    def zero(i):
        sl = pl.ds(i * _L, _L)
        hc[sl] = zi
        hs[sl] = zf

    return st[1], st[2], st[3], st[4], st[5], totc, tots


def _sc_body(x_hbm, out_hbm, row_a, row_b, h1c, h1s, h2c, h2s, h3c, h3s,
             res_v, sem_a, sem_b, *, n, k_above, dup_thresh, rows_per_w):
    wid = lax.axis_index("s") * _NC + lax.axis_index("c")
    iot = lax.iota(jnp.int32, _L)
    ones_i = jnp.full((_L,), 1, jnp.int32)
    zi = jnp.zeros((_L,), jnp.int32)
    zf = jnp.zeros((_L,), jnp.float32)
    res_s = jnp.zeros((_L,), jnp.float32)
    res_c = jnp.ones((_L,), jnp.float32)
    nchunks = n // _L
    row0 = wid * rows_per_w
    bufs = (row_a, row_b)
    sems = (sem_a, sem_b)

    @plsc.parallel_loop(0, _NB1 // _L, unroll=4)
    def z12(i):
        sl = pl.ds(i * _L, _L)
        h1c[sl] = zi
        h1s[sl] = zf
        h2c[sl] = zi
        h2s[sl] = zf

    @plsc.parallel_loop(0, _NB3 // _L, unroll=4)
    def z3(i):
        sl = pl.ds(i * _L, _L)
        h3c[sl] = zi
        h3s[sl] = zf

    copies = [pltpu.async_copy(x_hbm.at[row0], row_a, sem_a)]
    for j in range(rows_per_w):
        if j + 1 < rows_per_w:
            copies.append(pltpu.async_copy(
                x_hbm.at[row0 + j + 1], bufs[(j + 1) % 2], sems[(j + 1) % 2]))
        copies[j].wait()
        row_v = bufs[j % 2]

        @plsc.parallel_loop(0, nchunks, unroll=8)
        def p1(i):
            x = row_v[pl.ds(i * _L, _L)]
            key = _sortable_key(x)
            idx = lax.shift_right_arithmetic(key, 32 - _B1) + jnp.int32(_NB1 // 2)
            plsc.addupdate_scatter(h1c, [idx], ones_i)
            plsc.addupdate_scatter(h1s, [idx], x)

        a1 = n - k_above + 1
        b1, c1, s1, bc1, bs1, t1c, t1s = _scan_level(h1c, h1s, _NB1, a1)
        g_c = t1c - c1
        g_s = t1s - s1
        k2 = k_above - g_c
        top1 = b1 - jnp.int32(_NB1 // 2)

        @plsc.parallel_loop(0, nchunks, unroll=8)
        def p2(i):
            x = row_v[pl.ds(i * _L, _L)]
            key = _sortable_key(x)
            m = lax.shift_right_arithmetic(key, 32 - _B1) == top1
            idx = lax.shift_right_logical(key, 32 - _B1 - _B2) & jnp.int32(_NB2 - 1)
            plsc.addupdate_scatter(h2c, [idx], ones_i, mask=m)
            plsc.addupdate_scatter(h2s, [idx], x, mask=m)

        a2 = bc1 - k2 + 1
        b2, c2, s2, bc2, bs2, t2c, t2s = _scan_level(h2c, h2s, _NB2, a2)
        g_c = g_c + (t2c - c2)
        g_s = g_s + (t2s - s2)
        k3 = k2 - (t2c - c2)
        pref2 = (top1 << _B2) | b2

        @plsc.parallel_loop(0, nchunks, unroll=8)
        def p3(i):
            x = row_v[pl.ds(i * _L, _L)]
            key = _sortable_key(x)
            m = lax.shift_right_arithmetic(key, _B3) == pref2
            idx = key & jnp.int32(_NB3 - 1)
            plsc.addupdate_scatter(h3c, [idx], ones_i, mask=m)
            plsc.addupdate_scatter(h3s, [idx], x, mask=m)

        a3 = bc2 - k3 + 1
        b3, c3, s3, bc3, bs3, t3c, t3s = _scan_level(h3c, h3s, _NB3, a3)
        c_gt = g_c + (t3c - c3)
        s_gt = g_s + (t3s - s3)
        c_ge = c_gt + bc3
        s_ge = s_gt + bs3

        dup = c_ge >= dup_thresh
        cnt = jnp.where(dup, c_gt, c_ge).astype(jnp.float32)
        ssum = jnp.where(dup, s_gt, s_ge)
        res_s = jnp.where(iot == j, ssum, res_s)
        res_c = jnp.where(iot == j, cnt, res_c)

    res_v[...] = res_s / res_c
    pltpu.sync_copy(res_v, out_hbm.at[wid])


def _tc_body(x_ref, out_ref, *, k_above, n_above_lo):
    x = x_ref[...]
    bits = jax.lax.bitcast_convert_type(x, jnp.int32)
    key = bits ^ (lax.shift_right_arithmetic(bits, 31) & jnp.int32(0x7FFFFFFF))
    t0 = jnp.full((x.shape[0], 1), jnp.int32(-(2 ** 31)), dtype=jnp.int32)

    def step(i, t):
        cand = t + jnp.left_shift(jnp.int32(1), jnp.int32(31) - i)
        cnt = jnp.sum((key >= cand).astype(jnp.int32), axis=1, keepdims=True)
        return jnp.where(cnt >= k_above, cand, t)

    t = lax.fori_loop(0, 32, step, t0)
    ge = key >= t
    gt = key > t
    c_ge = jnp.sum(ge.astype(jnp.int32), axis=1, keepdims=True)
    c_gt = jnp.sum(gt.astype(jnp.int32), axis=1, keepdims=True)
    s_ge = jnp.sum(jnp.where(ge, x, 0.0), axis=1, keepdims=True)
    s_gt = jnp.sum(jnp.where(gt, x, 0.0), axis=1, keepdims=True)
    dup = c_ge >= n_above_lo
    cnt = jnp.where(dup, c_gt, c_ge).astype(jnp.float32)
    s = jnp.where(dup, s_gt, s_ge)
    out_ref[...] = s / cnt


_SC_ROWS = 64


@jax.jit
def kernel(patch_logits):
    b, n = patch_logits.shape
    q = (100 - 10) / 100.0
    i0 = int(q * (n - 1))
    k_above = n - i0 - 1
    nw = _NC * _NS
    rows_per_w = _SC_ROWS // nw

    body = functools.partial(
        _sc_body, n=n, k_above=k_above, dup_thresh=n - i0,
        rows_per_w=rows_per_w)
    out_sc = pl.kernel(
        body,
        out_type=jax.ShapeDtypeStruct((nw, _L), jnp.float32),
        mesh=plsc.VectorSubcoreMesh(core_axis_name="c", subcore_axis_name="s",
                                    num_cores=_NC, num_subcores=_NS),
        compiler_params=pltpu.CompilerParams(needs_layout_passes=False),
        scratch_types=[
            pltpu.VMEM((n,), jnp.float32),
            pltpu.VMEM((n,), jnp.float32),
            pltpu.VMEM((_NB1,), jnp.int32),
            pltpu.VMEM((_NB1,), jnp.float32),
            pltpu.VMEM((_NB2,), jnp.int32),
            pltpu.VMEM((_NB2,), jnp.float32),
            pltpu.VMEM((_NB3,), jnp.int32),
            pltpu.VMEM((_NB3,), jnp.float32),
            pltpu.VMEM((_L,), jnp.float32),
            pltpu.SemaphoreType.DMA,
            pltpu.SemaphoreType.DMA,
        ],
    )(patch_logits)
    out_sc = out_sc[:, :rows_per_w].reshape(_SC_ROWS, 1)

    tc_rows = b - _SC_ROWS
    rows_per_block = 16
    blk0 = _SC_ROWS // rows_per_block
    out_tc = pl.pallas_call(
        functools.partial(_tc_body, k_above=k_above, n_above_lo=n - i0),
        grid=(tc_rows // rows_per_block,),
        in_specs=[pl.BlockSpec((rows_per_block, n), lambda i: (i + blk0, 0))],
        out_specs=pl.BlockSpec((rows_per_block, 1), lambda i: (i, 0)),
        out_shape=jax.ShapeDtypeStruct((tc_rows, 1), jnp.float32),
    )(patch_logits)

    return jnp.concatenate([out_sc, out_tc], axis=0)
